# baseline (speedup 1.0000x reference)
import functools

import jax
import jax.numpy as jnp
from jax import lax
from jax.experimental import pallas as pl
from jax.experimental.pallas import tpu as pltpu
from jax.experimental.pallas import tpu_sc as plsc

N = 10000
E = 320000
D = 128

NC = 2
NS = 16
NW = NC * NS

CH = 128
NCHUNKS = 2560
EPAD = NCHUNKS * CH
K0 = 116
K1 = 44
KD = 160
KDH = 80

RPT = 640
RPAD = RPT * NS
DPT = 640
DPAD = DPT * NS
PAD_DST = N
PKT = 320
ISH = 14
IMASK = (1 << ISH) - 1

ROWS_B = 1000


def _unpack(pidx, j, buf, shift, mask):
    for t in range(CH // 16):
        v = pidx[pl.ds(j * CH + t * 16, 16)]
        buf[pl.ds(t * 16, 16)] = (v >> shift) & mask


def _run_chunks(kc, y, pidx, rows0, rows1, sbuf0, sbuf1, dbuf,
                sem0, sem1, agg):
    _unpack(pidx, 0, sbuf0, 0, IMASK)
    pltpu.async_copy(y.at[sbuf0], rows0, sem0)

    def _pair(i, carry):
        j0 = 2 * i
        j1 = 2 * i + 1
        _unpack(pidx, j1, sbuf1, 0, IMASK)
        pltpu.make_async_copy(y.at[sbuf0], rows0, sem0).wait()
        pltpu.async_copy(y.at[sbuf1], rows1, sem1)

        _unpack(pidx, j0, dbuf, ISH, IMASK)
        pltpu.sync_copy(rows0, agg.at[dbuf], add=True)

        @pl.when(i < kc // 2 - 1)
        def _():
            _unpack(pidx, j0 + 2, sbuf0, 0, IMASK)

        pltpu.make_async_copy(y.at[sbuf1], rows1, sem1).wait()

        @pl.when(i < kc // 2 - 1)
        def _():
            pltpu.async_copy(y.at[sbuf0], rows0, sem0)

        _unpack(pidx, j1, dbuf, ISH, IMASK)
        pltpu.sync_copy(rows1, agg.at[dbuf], add=True)
        return carry
    lax.fori_loop(0, kc // 2, _pair, 0)


def _zero_agg_slice(rows0, agg, s):
    z16 = jnp.zeros((16,), jnp.float32)

    def _zrow(i, carry):
        def _zcol(j, carry2):
            rows0[i, pl.ds(j * 16, 16)] = z16
            return carry2
        return lax.fori_loop(0, D // 16, _zcol, carry)
    lax.fori_loop(0, CH, _zrow, 0)

    base = s * RPT
    for k in range(RPT // CH):
        pltpu.sync_copy(rows0, agg.at[pl.ds(base + k * CH, CH)])
    rem = RPT - (RPT // CH) * CH
    if rem:
        pltpu.sync_copy(rows0.at[pl.ds(0, rem)],
                        agg.at[pl.ds(base + RPT - rem, rem)])


def _sc_agg_body(y, pidxh, p0, p1, dcnt, pidx, rows0, rows1, sbuf0, sbuf1,
                 dbuf, ones_v, sem0, sem1, agg, deg):
    c = lax.axis_index("c")
    s = lax.axis_index("s")
    base = s * RPT

    @pl.when(c == 0)
    def _():
        _zero_agg_slice(rows0, agg, s)
        plsc.subcore_barrier()

        pltpu.sync_copy(pidxh.at[pl.ds(s * K0 * CH, K0 * CH)],
                        pidx.at[pl.ds(0, K0 * CH)])
        _run_chunks(K0, y, pidx, rows0, rows1, sbuf0, sbuf1, dbuf,
                    sem0, sem1, agg)

        plsc.subcore_barrier()

        pltpu.sync_copy(agg.at[pl.ds(base, RPT)], p0.at[pl.ds(base, RPT)])

    @pl.when(c == 1)
    def _():
        z16 = jnp.zeros((16,), jnp.float32)
        for j in range(CH // 16):
            ones_v[pl.ds(j * 16, 16)] = z16 + 1.0

        _zero_agg_slice(rows0, agg, s)
        for k in range(DPT // CH):
            pltpu.sync_copy(rows0.at[0], deg.at[pl.ds(s * DPT + k * CH, CH)])

        plsc.subcore_barrier()

        def _dhalf(h, carry):
            pltpu.sync_copy(
                pidxh.at[pl.ds((s * KD + h * KDH) * CH, KDH * CH)],
                pidx.at[pl.ds(0, KDH * CH)])

            def _chunk(j, carry2):
                _unpack(pidx, j, dbuf, ISH, IMASK)
                pltpu.sync_copy(ones_v, deg.at[dbuf], add=True)
                return carry2
            lax.fori_loop(0, KDH, _chunk, 0)
            return carry
        lax.fori_loop(0, KD // KDH, _dhalf, 0)

        pltpu.sync_copy(
            pidxh.at[pl.ds((NS * K0 + s * K1) * CH, K1 * CH)],
            pidx.at[pl.ds(0, K1 * CH)])
        _run_chunks(K1, y, pidx, rows0, rows1, sbuf0, sbuf1, dbuf,
                    sem0, sem1, agg)

        plsc.subcore_barrier()

        def _cvt(g, carry):
            rbase = base + g * CH
            pltpu.sync_copy(agg.at[pl.ds(rbase, CH)], rows0)

            def _pk(i, carry2):
                r = i >> 1
                cbase = (i & 1) * (D // 2)
                for t in range(D // 32):
                    a = rows0[i, pl.ds(t * 32, 16)]
                    b = rows0[i, pl.ds(t * 32 + 16, 16)]
                    ba = lax.bitcast_convert_type(a, jnp.int32)
                    ra = ba + jnp.int32(0x7FFF) + ((ba >> 16) & 1)
                    bb = lax.bitcast_convert_type(b, jnp.int32)
                    rb = bb + jnp.int32(0x7FFF) + ((bb >> 16) & 1)
                    w = (((ra >> 16) & jnp.int32(0xFFFF))
                         | (rb & jnp.int32(-65536)))
                    rows1[r, pl.ds(cbase + t * 16, 16)] = (
                        lax.bitcast_convert_type(w, jnp.float32))
                return carry2
            lax.fori_loop(0, CH, _pk, 0)

            pltpu.sync_copy(rows1.at[pl.ds(0, CH // 2)],
                            agg.at[pl.ds(base + g * (CH // 2), CH // 2)])
            return carry
        lax.fori_loop(0, RPT // CH, _cvt, 0)
        pltpu.sync_copy(agg.at[pl.ds(base, RPT // 2)],
                        p1.at[pl.ds(s * PKT, RPT // 2)])

        pltpu.sync_copy(deg.at[pl.ds(s * DPT, DPT)],
                        dcnt.at[pl.ds(s * DPT, DPT)])


def _make_sc_agg():
    mesh = plsc.VectorSubcoreMesh(core_axis_name="c", subcore_axis_name="s",
                                  num_cores=NC, num_subcores=NS)
    out_type = (jax.ShapeDtypeStruct((RPAD, D), jnp.float32),
                jax.ShapeDtypeStruct((NS * PKT, D), jnp.float32),
                jax.ShapeDtypeStruct((DPAD,), jnp.float32))
    return pl.kernel(
        _sc_agg_body,
        out_type=out_type,
        mesh=mesh,
        scratch_types=[
            pltpu.VMEM((K0 * CH,), jnp.int32),
            pltpu.VMEM((CH, D), jnp.float32),
            pltpu.VMEM((CH, D), jnp.float32),
            pltpu.VMEM((CH,), jnp.int32),
            pltpu.VMEM((CH,), jnp.int32),
            pltpu.VMEM((CH,), jnp.int32),
            pltpu.VMEM((CH,), jnp.float32),
            pltpu.SemaphoreType.DMA,
            pltpu.SemaphoreType.DMA,
            pltpu.VMEM_SHARED((RPAD, D), jnp.float32),
            pltpu.VMEM_SHARED((DPAD,), jnp.float32),
        ],
    )


def _mm_body(x_ref, w_ref, o_ref):
    o_ref[...] = jnp.dot(x_ref[...], w_ref[...],
                         preferred_element_type=jnp.float32)


def _tc_mm(x, w):
    return pl.pallas_call(
        _mm_body,
        grid=(N // ROWS_B,),
        in_specs=[pl.BlockSpec((ROWS_B, D), lambda i: (i, 0)),
                  pl.BlockSpec((D, D), lambda i: (0, 0))],
        out_specs=pl.BlockSpec((ROWS_B, D), lambda i: (i, 0)),
        out_shape=jax.ShapeDtypeStruct((N, D), jnp.float32),
    )(x, w)


def _norm_mm_body(p0_ref, p1_ref, d_ref, b_ref, w_ref, y_ref, r_ref):
    r = 1.0 / jnp.maximum(d_ref[...], 1.0)
    pp = p0_ref[...] + p1_ref[...].astype(jnp.float32)
    h = pp * r + b_ref[...]
    y_ref[...] = jnp.dot(h, w_ref[...], preferred_element_type=jnp.float32)
    r_ref[...] = r


def _tc_norm_mm(p0, p1, d, b, w):
    return pl.pallas_call(
        _norm_mm_body,
        grid=(N // ROWS_B,),
        in_specs=[pl.BlockSpec((ROWS_B, D), lambda i: (i, 0)),
                  pl.BlockSpec((ROWS_B, D), lambda i: (i, 0)),
                  pl.BlockSpec((ROWS_B, 1), lambda i: (i, 0)),
                  pl.BlockSpec((1, D), lambda i: (0, 0)),
                  pl.BlockSpec((D, D), lambda i: (0, 0))],
        out_specs=(pl.BlockSpec((ROWS_B, D), lambda i: (i, 0)),
                   pl.BlockSpec((ROWS_B, 1), lambda i: (i, 0))),
        out_shape=(jax.ShapeDtypeStruct((N, D), jnp.float32),
                   jax.ShapeDtypeStruct((N, 1), jnp.float32)),
    )(p0, p1, d, b, w)


def _final_body(q0_ref, q1_ref, r_ref, b_ref, o_ref):
    qq = q0_ref[...] + q1_ref[...].astype(jnp.float32)
    o_ref[...] = qq * r_ref[...] + b_ref[...]


def _tc_final(q0, q1, r, b):
    return pl.pallas_call(
        _final_body,
        grid=(N // ROWS_B,),
        in_specs=[pl.BlockSpec((ROWS_B, D), lambda i: (i, 0)),
                  pl.BlockSpec((ROWS_B, D), lambda i: (i, 0)),
                  pl.BlockSpec((ROWS_B, 1), lambda i: (i, 0)),
                  pl.BlockSpec((1, D), lambda i: (0, 0))],
        out_specs=pl.BlockSpec((ROWS_B, D), lambda i: (i, 0)),
        out_shape=jax.ShapeDtypeStruct((N, D), jnp.float32),
    )(q0, q1, r, b)


def _decode_bf16_pairs(pw):
    b = jax.lax.bitcast_convert_type(pw, jnp.bfloat16)
    v = b.reshape(NS, PKT, 2, D // 32, 16, 2)
    v = v.transpose(0, 1, 2, 3, 5, 4)
    v = v.reshape(NS, PKT, 2, D)[:, :RPT // 2]
    return v.reshape(NS * RPT, D)


def kernel(x, edge_index, W1, b1, W2, b2):
    src = edge_index[0].astype(jnp.int32)
    dst = edge_index[1].astype(jnp.int32)
    packed = src | (dst << ISH)
    pidx = jnp.concatenate(
        [packed, jnp.full((EPAD - E,), PAD_DST << ISH, jnp.int32)])

    sc_agg = _make_sc_agg()

    y1 = _tc_mm(x, W1)
    p0, p1w, dcnt = sc_agg(y1, pidx)

    y2, rdeg = _tc_norm_mm(p0[:N], _decode_bf16_pairs(p1w)[:N],
                           dcnt[:N].reshape(N, 1), b1.reshape(1, D), W2)

    q0, q1w, _ = sc_agg(y2, pidx)
    return _tc_final(q0[:N], _decode_bf16_pairs(q1w)[:N], rdeg,
                     b2.reshape(1, D))


# speedup vs baseline: 1.0821x; 1.0821x over previous
import functools

import jax
import jax.numpy as jnp
from jax import lax
from jax.experimental import pallas as pl
from jax.experimental.pallas import tpu as pltpu
from jax.experimental.pallas import tpu_sc as plsc

N = 10000
E = 320000
D = 128

NC = 2
NS = 16
NW = NC * NS

CH = 128
NCHUNKS = 2560
EPAD = NCHUNKS * CH
K0 = 120
K1 = 40
KD = 160
KDH = 80

RPT = 632
RPAD = RPT * NS
DPT = 640
DPAD = DPT * NS
PAD_DST = N
ISH = 14
IMASK = (1 << ISH) - 1

ROWS_B = 1000


def _unpack(pidx, j, buf, shift, mask):
    for t in range(CH // 16):
        v = pidx[pl.ds(j * CH + t * 16, 16)]
        buf[pl.ds(t * 16, 16)] = (v >> shift) & mask


def _run_chunks(kc, y, pidx, rows0, rows1, sbuf0, sbuf1, dbuf,
                sem0, sem1, agg):
    _unpack(pidx, 0, sbuf0, 0, IMASK)
    pltpu.async_copy(y.at[sbuf0], rows0, sem0)

    def _pair(i, carry):
        j0 = 2 * i
        j1 = 2 * i + 1
        _unpack(pidx, j1, sbuf1, 0, IMASK)
        pltpu.make_async_copy(y.at[sbuf0], rows0, sem0).wait()
        pltpu.async_copy(y.at[sbuf1], rows1, sem1)

        _unpack(pidx, j0, dbuf, ISH, IMASK)
        pltpu.sync_copy(rows0, agg.at[dbuf], add=True)

        @pl.when(i < kc // 2 - 1)
        def _():
            _unpack(pidx, j0 + 2, sbuf0, 0, IMASK)

        pltpu.make_async_copy(y.at[sbuf1], rows1, sem1).wait()

        @pl.when(i < kc // 2 - 1)
        def _():
            pltpu.async_copy(y.at[sbuf0], rows0, sem0)

        _unpack(pidx, j1, dbuf, ISH, IMASK)
        pltpu.sync_copy(rows1, agg.at[dbuf], add=True)
        return carry
    lax.fori_loop(0, kc // 2, _pair, 0)


def _zero_agg_slice(rows0, agg, s):
    z16 = jnp.zeros((16,), jnp.float32)

    def _zrow(i, carry):
        def _zcol(j, carry2):
            rows0[i, pl.ds(j * 16, 16)] = z16
            return carry2
        return lax.fori_loop(0, D // 16, _zcol, carry)
    lax.fori_loop(0, CH, _zrow, 0)

    base = s * RPT
    for k in range(RPT // CH):
        pltpu.sync_copy(rows0, agg.at[pl.ds(base + k * CH, CH)])
    rem = RPT - (RPT // CH) * CH
    if rem:
        pltpu.sync_copy(rows0.at[pl.ds(0, rem)],
                        agg.at[pl.ds(base + RPT - rem, rem)])


def _sc_agg_body(y, pidxh, p0, p1, dcnt, pidx, rows0, rows1, sbuf0, sbuf1,
                 dbuf, ones_v, sem0, sem1, agg, deg):
    c = lax.axis_index("c")
    s = lax.axis_index("s")
    base = s * RPT

    @pl.when(c == 0)
    def _():
        _zero_agg_slice(rows0, agg, s)
        plsc.subcore_barrier()

        pltpu.sync_copy(pidxh.at[pl.ds(s * K0 * CH, K0 * CH)],
                        pidx.at[pl.ds(0, K0 * CH)])
        _run_chunks(K0, y, pidx, rows0, rows1, sbuf0, sbuf1, dbuf,
                    sem0, sem1, agg)

        plsc.subcore_barrier()

        pltpu.sync_copy(agg.at[pl.ds(base, RPT)], p0.at[pl.ds(base, RPT)])

    @pl.when(c == 1)
    def _():
        z16 = jnp.zeros((16,), jnp.float32)
        for j in range(CH // 16):
            ones_v[pl.ds(j * 16, 16)] = z16 + 1.0

        _zero_agg_slice(rows0, agg, s)
        for k in range(DPT // CH):
            pltpu.sync_copy(rows0.at[0], deg.at[pl.ds(s * DPT + k * CH, CH)])

        plsc.subcore_barrier()

        def _dhalf(h, carry):
            pltpu.sync_copy(
                pidxh.at[pl.ds((s * KD + h * KDH) * CH, KDH * CH)],
                pidx.at[pl.ds(0, KDH * CH)])

            def _chunk(j, carry2):
                _unpack(pidx, j, dbuf, ISH, IMASK)
                pltpu.sync_copy(ones_v, deg.at[dbuf], add=True)
                return carry2
            lax.fori_loop(0, KDH, _chunk, 0)
            return carry
        lax.fori_loop(0, KD // KDH, _dhalf, 0)

        pltpu.sync_copy(
            pidxh.at[pl.ds((NS * K0 + s * K1) * CH, K1 * CH)],
            pidx.at[pl.ds(0, K1 * CH)])
        _run_chunks(K1, y, pidx, rows0, rows1, sbuf0, sbuf1, dbuf,
                    sem0, sem1, agg)

        plsc.subcore_barrier()

        pltpu.sync_copy(agg.at[pl.ds(base, RPT)], p1.at[pl.ds(base, RPT)])

        pltpu.sync_copy(deg.at[pl.ds(s * DPT, DPT)],
                        dcnt.at[pl.ds(s * DPT, DPT)])


def _make_sc_agg():
    mesh = plsc.VectorSubcoreMesh(core_axis_name="c", subcore_axis_name="s",
                                  num_cores=NC, num_subcores=NS)
    out_type = (jax.ShapeDtypeStruct((RPAD, D), jnp.float32),
                jax.ShapeDtypeStruct((RPAD, D), jnp.float32),
                jax.ShapeDtypeStruct((DPAD,), jnp.float32))
    return pl.kernel(
        _sc_agg_body,
        out_type=out_type,
        mesh=mesh,
        scratch_types=[
            pltpu.VMEM((K0 * CH,), jnp.int32),
            pltpu.VMEM((CH, D), jnp.float32),
            pltpu.VMEM((CH, D), jnp.float32),
            pltpu.VMEM((CH,), jnp.int32),
            pltpu.VMEM((CH,), jnp.int32),
            pltpu.VMEM((CH,), jnp.int32),
            pltpu.VMEM((CH,), jnp.float32),
            pltpu.SemaphoreType.DMA,
            pltpu.SemaphoreType.DMA,
            pltpu.VMEM_SHARED((RPAD, D), jnp.float32),
            pltpu.VMEM_SHARED((DPAD,), jnp.float32),
        ],
    )


def _mm_body(x_ref, w_ref, o_ref):
    o_ref[...] = jnp.dot(x_ref[...], w_ref[...],
                         preferred_element_type=jnp.float32)


def _tc_mm(x, w):
    return pl.pallas_call(
        _mm_body,
        grid=(N // ROWS_B,),
        in_specs=[pl.BlockSpec((ROWS_B, D), lambda i: (i, 0)),
                  pl.BlockSpec((D, D), lambda i: (0, 0))],
        out_specs=pl.BlockSpec((ROWS_B, D), lambda i: (i, 0)),
        out_shape=jax.ShapeDtypeStruct((N, D), jnp.float32),
    )(x, w)


def _norm_mm_body(p0_ref, p1_ref, d_ref, b_ref, w_ref, y_ref, r_ref):
    r = 1.0 / jnp.maximum(d_ref[...], 1.0)
    pp = p0_ref[...] + p1_ref[...].astype(jnp.float32)
    h = pp * r + b_ref[...]
    y_ref[...] = jnp.dot(h, w_ref[...], preferred_element_type=jnp.float32)
    r_ref[...] = r


def _tc_norm_mm(p0, p1, d, b, w):
    return pl.pallas_call(
        _norm_mm_body,
        grid=(N // ROWS_B,),
        in_specs=[pl.BlockSpec((ROWS_B, D), lambda i: (i, 0)),
                  pl.BlockSpec((ROWS_B, D), lambda i: (i, 0)),
                  pl.BlockSpec((ROWS_B, 1), lambda i: (i, 0)),
                  pl.BlockSpec((1, D), lambda i: (0, 0)),
                  pl.BlockSpec((D, D), lambda i: (0, 0))],
        out_specs=(pl.BlockSpec((ROWS_B, D), lambda i: (i, 0)),
                   pl.BlockSpec((ROWS_B, 1), lambda i: (i, 0))),
        out_shape=(jax.ShapeDtypeStruct((N, D), jnp.float32),
                   jax.ShapeDtypeStruct((N, 1), jnp.float32)),
    )(p0, p1, d, b, w)


def _final_body(q0_ref, q1_ref, r_ref, b_ref, o_ref):
    qq = q0_ref[...] + q1_ref[...].astype(jnp.float32)
    o_ref[...] = qq * r_ref[...] + b_ref[...]


def _tc_final(q0, q1, r, b):
    return pl.pallas_call(
        _final_body,
        grid=(N // ROWS_B,),
        in_specs=[pl.BlockSpec((ROWS_B, D), lambda i: (i, 0)),
                  pl.BlockSpec((ROWS_B, D), lambda i: (i, 0)),
                  pl.BlockSpec((ROWS_B, 1), lambda i: (i, 0)),
                  pl.BlockSpec((1, D), lambda i: (0, 0))],
        out_specs=pl.BlockSpec((ROWS_B, D), lambda i: (i, 0)),
        out_shape=jax.ShapeDtypeStruct((N, D), jnp.float32),
    )(q0, q1, r, b)


def kernel(x, edge_index, W1, b1, W2, b2):
    src = edge_index[0].astype(jnp.int32)
    dst = edge_index[1].astype(jnp.int32)
    packed = src | (dst << ISH)
    pidx = jnp.concatenate(
        [packed, jnp.full((EPAD - E,), PAD_DST << ISH, jnp.int32)])

    sc_agg = _make_sc_agg()

    y1 = _tc_mm(x, W1)
    p0, p1w, dcnt = sc_agg(y1, pidx)

    y2, rdeg = _tc_norm_mm(p0[:N], p1w[:N],
                           dcnt[:N].reshape(N, 1), b1.reshape(1, D), W2)

    q0, q1w, _ = sc_agg(y2, pidx)
    return _tc_final(q0[:N], q1w[:N], rdeg, b2.reshape(1, D))


# speedup vs baseline: 1.2432x; 1.1489x over previous
import functools

import jax
import jax.numpy as jnp
from jax import lax
from jax.experimental import pallas as pl
from jax.experimental.pallas import tpu as pltpu
from jax.experimental.pallas import tpu_sc as plsc

N = 10000
E = 320000
D = 128

NC = 2
NS = 16
NW = NC * NS

CH = 128
NCHUNKS = 2560
EPAD = NCHUNKS * CH
K0 = 120
K1 = 40

RPT = 632
RPAD = RPT * NS
DPT = 640
DPAD = DPT * NS
PAD_DST = N
ISH = 14
IMASK = (1 << ISH) - 1

ROWS_B = 1000


def _unpack(pidx, j, buf, shift, mask):
    for t in range(CH // 16):
        v = pidx[pl.ds(j * CH + t * 16, 16)]
        buf[pl.ds(t * 16, 16)] = (v >> shift) & mask


def _run_chunks(kc, y, pidx, rows0, rows1, sbuf0, sbuf1, dbuf, ones_v,
                sem0, sem1, agg, deg):
    _unpack(pidx, 0, sbuf0, 0, IMASK)
    pltpu.async_copy(y.at[sbuf0], rows0, sem0)

    def _pair(i, carry):
        j0 = 2 * i
        j1 = 2 * i + 1
        _unpack(pidx, j1, sbuf1, 0, IMASK)
        pltpu.make_async_copy(y.at[sbuf0], rows0, sem0).wait()
        pltpu.async_copy(y.at[sbuf1], rows1, sem1)

        _unpack(pidx, j0, dbuf, ISH, IMASK)
        pltpu.sync_copy(rows0, agg.at[dbuf], add=True)
        pltpu.sync_copy(ones_v, deg.at[dbuf], add=True)

        @pl.when(i < kc // 2 - 1)
        def _():
            _unpack(pidx, j0 + 2, sbuf0, 0, IMASK)

        pltpu.make_async_copy(y.at[sbuf1], rows1, sem1).wait()

        @pl.when(i < kc // 2 - 1)
        def _():
            pltpu.async_copy(y.at[sbuf0], rows0, sem0)

        _unpack(pidx, j1, dbuf, ISH, IMASK)
        pltpu.sync_copy(rows1, agg.at[dbuf], add=True)
        pltpu.sync_copy(ones_v, deg.at[dbuf], add=True)
        return carry
    lax.fori_loop(0, kc // 2, _pair, 0)


def _zero_agg_slice(rows0, agg, s):
    z16 = jnp.zeros((16,), jnp.float32)

    def _zrow(i, carry):
        def _zcol(j, carry2):
            rows0[i, pl.ds(j * 16, 16)] = z16
            return carry2
        return lax.fori_loop(0, D // 16, _zcol, carry)
    lax.fori_loop(0, CH, _zrow, 0)

    base = s * RPT
    for k in range(RPT // CH):
        pltpu.sync_copy(rows0, agg.at[pl.ds(base + k * CH, CH)])
    rem = RPT - (RPT // CH) * CH
    if rem:
        pltpu.sync_copy(rows0.at[pl.ds(0, rem)],
                        agg.at[pl.ds(base + RPT - rem, rem)])


def _sc_agg_body(y, pidxh, p, dcnt, pidx, rows0, rows1, sbuf0, sbuf1,
                 dbuf, ones_v, sem0, sem1, agg, deg):
    c = lax.axis_index("c")
    s = lax.axis_index("s")
    base = s * RPT

    z16 = jnp.zeros((16,), jnp.float32)
    for j in range(CH // 16):
        ones_v[pl.ds(j * 16, 16)] = z16 + 1.0

    _zero_agg_slice(rows0, agg, s)
    for k in range(DPT // CH):
        pltpu.sync_copy(rows0.at[0], deg.at[pl.ds(s * DPT + k * CH, CH)])

    @pl.when(c == 0)
    def _():
        pltpu.sync_copy(pidxh.at[pl.ds(s * K0 * CH, K0 * CH)],
                        pidx.at[pl.ds(0, K0 * CH)])

    @pl.when(c == 1)
    def _():
        pltpu.sync_copy(
            pidxh.at[pl.ds((NS * K0 + s * K1) * CH, K1 * CH)],
            pidx.at[pl.ds(0, K1 * CH)])

    plsc.subcore_barrier()

    args = (y, pidx, rows0, rows1, sbuf0, sbuf1, dbuf, ones_v,
            sem0, sem1, agg, deg)

    @pl.when(c == 0)
    def _():
        _run_chunks(K0, *args)

    @pl.when(c == 1)
    def _():
        _run_chunks(K1, *args)

    plsc.subcore_barrier()

    pltpu.sync_copy(agg.at[pl.ds(base, RPT)],
                    p.at[pl.ds(c * RPAD + base, RPT)])
    pltpu.sync_copy(deg.at[pl.ds(s * DPT, DPT)],
                    dcnt.at[pl.ds(c * DPAD + s * DPT, DPT)])


def _make_sc_agg():
    mesh = plsc.VectorSubcoreMesh(core_axis_name="c", subcore_axis_name="s",
                                  num_cores=NC, num_subcores=NS)
    out_type = (jax.ShapeDtypeStruct((NC * RPAD, D), jnp.float32),
                jax.ShapeDtypeStruct((NC * DPAD,), jnp.float32))
    return pl.kernel(
        _sc_agg_body,
        out_type=out_type,
        mesh=mesh,
        scratch_types=[
            pltpu.VMEM((K0 * CH,), jnp.int32),
            pltpu.VMEM((CH, D), jnp.float32),
            pltpu.VMEM((CH, D), jnp.float32),
            pltpu.VMEM((CH,), jnp.int32),
            pltpu.VMEM((CH,), jnp.int32),
            pltpu.VMEM((CH,), jnp.int32),
            pltpu.VMEM((CH,), jnp.float32),
            pltpu.SemaphoreType.DMA,
            pltpu.SemaphoreType.DMA,
            pltpu.VMEM_SHARED((RPAD, D), jnp.float32),
            pltpu.VMEM_SHARED((DPAD,), jnp.float32),
        ],
    )


def _mm_body(x_ref, w_ref, o_ref):
    o_ref[...] = jnp.dot(x_ref[...], w_ref[...],
                         preferred_element_type=jnp.float32)


def _tc_mm(x, w):
    return pl.pallas_call(
        _mm_body,
        grid=(N // ROWS_B,),
        in_specs=[pl.BlockSpec((ROWS_B, D), lambda i: (i, 0)),
                  pl.BlockSpec((D, D), lambda i: (0, 0))],
        out_specs=pl.BlockSpec((ROWS_B, D), lambda i: (i, 0)),
        out_shape=jax.ShapeDtypeStruct((N, D), jnp.float32),
    )(x, w)


def _norm_mm_body(p0_ref, p1_ref, d0_ref, d1_ref, b_ref, w_ref, y_ref, r_ref):
    r = 1.0 / jnp.maximum(d0_ref[...] + d1_ref[...], 1.0)
    h = (p0_ref[...] + p1_ref[...]) * r + b_ref[...]
    y_ref[...] = jnp.dot(h, w_ref[...], preferred_element_type=jnp.float32)
    r_ref[...] = r


def _tc_norm_mm(p0, p1, d0, d1, b, w):
    return pl.pallas_call(
        _norm_mm_body,
        grid=(N // ROWS_B,),
        in_specs=[pl.BlockSpec((ROWS_B, D), lambda i: (i, 0)),
                  pl.BlockSpec((ROWS_B, D), lambda i: (i, 0)),
                  pl.BlockSpec((ROWS_B, 1), lambda i: (i, 0)),
                  pl.BlockSpec((ROWS_B, 1), lambda i: (i, 0)),
                  pl.BlockSpec((1, D), lambda i: (0, 0)),
                  pl.BlockSpec((D, D), lambda i: (0, 0))],
        out_specs=(pl.BlockSpec((ROWS_B, D), lambda i: (i, 0)),
                   pl.BlockSpec((ROWS_B, 1), lambda i: (i, 0))),
        out_shape=(jax.ShapeDtypeStruct((N, D), jnp.float32),
                   jax.ShapeDtypeStruct((N, 1), jnp.float32)),
    )(p0, p1, d0, d1, b, w)


def _final_body(q0_ref, q1_ref, r_ref, b_ref, o_ref):
    o_ref[...] = (q0_ref[...] + q1_ref[...]) * r_ref[...] + b_ref[...]


def _tc_final(q0, q1, r, b):
    return pl.pallas_call(
        _final_body,
        grid=(N // ROWS_B,),
        in_specs=[pl.BlockSpec((ROWS_B, D), lambda i: (i, 0)),
                  pl.BlockSpec((ROWS_B, D), lambda i: (i, 0)),
                  pl.BlockSpec((ROWS_B, 1), lambda i: (i, 0)),
                  pl.BlockSpec((1, D), lambda i: (0, 0))],
        out_specs=pl.BlockSpec((ROWS_B, D), lambda i: (i, 0)),
        out_shape=jax.ShapeDtypeStruct((N, D), jnp.float32),
    )(q0, q1, r, b)


def kernel(x, edge_index, W1, b1, W2, b2):
    src = edge_index[0].astype(jnp.int32)
    dst = edge_index[1].astype(jnp.int32)
    packed = src | (dst << ISH)
    pidx = jnp.concatenate(
        [packed, jnp.full((EPAD - E,), PAD_DST << ISH, jnp.int32)])

    sc_agg = _make_sc_agg()

    y1 = _tc_mm(x, W1)
    p, dcnt = sc_agg(y1, pidx)
    d0 = dcnt[:N].reshape(N, 1)
    d1 = dcnt[DPAD:DPAD + N].reshape(N, 1)

    y2, rdeg = _tc_norm_mm(p[:N], p[RPAD:RPAD + N], d0, d1,
                           b1.reshape(1, D), W2)

    q, _ = sc_agg(y2, pidx)
    return _tc_final(q[:N], q[RPAD:RPAD + N], rdeg, b2.reshape(1, D))
